# baseline (speedup 1.0000x reference)
import functools

import jax
import jax.numpy as jnp
from jax import lax
from jax.experimental import pallas as pl
from jax.experimental.pallas import tpu as pltpu
from jax.experimental.pallas import tpu_sc as plsc

N = 10000
D = 128
E = 320000

NC = 2
NS = 16
NW = NC * NS
CHUNK = 80
ITERS = 128
EPW = ITERS * CHUNK
RPT = 632
NP = RPT * NS


def _spmm_sc_body(h_hbm, ei_hbm, zeros_hbm, out_hbm,
                  src_v, dst_v, rows_v, acc_sh, sem):
    c = lax.axis_index("c")
    s = lax.axis_index("s")
    wid = c * NS + s
    rbase = s * RPT

    pltpu.sync_copy(zeros_hbm, acc_sh.at[pl.ds(rbase, RPT)])
    pltpu.sync_copy(ei_hbm.at[0, wid], src_v)
    pltpu.sync_copy(ei_hbm.at[1, wid], dst_v)
    plsc.subcore_barrier()

    def body(i, carry):
        pltpu.async_copy(h_hbm.at[src_v.at[i]], rows_v, sem).wait()
        return carry

    lax.fori_loop(0, ITERS, body, 0)
    plsc.subcore_barrier()
    pltpu.sync_copy(acc_sh.at[pl.ds(rbase, RPT)],
                    out_hbm.at[c, pl.ds(rbase, RPT)])


_spmm_sc = pl.kernel(
    _spmm_sc_body,
    out_type=jax.ShapeDtypeStruct((NC, NP, D), jnp.float32),
    mesh=plsc.VectorSubcoreMesh(core_axis_name="c", subcore_axis_name="s"),
    scratch_types=[
        pltpu.VMEM((ITERS, CHUNK), jnp.int32),
        pltpu.VMEM((ITERS, CHUNK), jnp.int32),
        pltpu.VMEM((CHUNK, D), jnp.float32),
        pltpu.VMEM_SHARED((NP, D), jnp.float32),
        pltpu.SemaphoreType.DMA,
    ],
)


def _bn_relu(h, g, b):
    mean = jnp.mean(h, axis=0, keepdims=True)
    var = jnp.mean((h - mean) ** 2, axis=0, keepdims=True)
    return jnp.maximum((h - mean) * lax.rsqrt(var + 1e-5) * g + b, 0.0)


def _stage1_body(x_ref, p0_ref, p1_ref, wa_ref, wb_ref, b_ref, g_ref,
                 beta_ref, o_ref):
    agg = p0_ref[...] + p1_ref[...]
    h = (jnp.dot(x_ref[...], wa_ref[...], preferred_element_type=jnp.float32)
         + jnp.dot(agg, wb_ref[...], preferred_element_type=jnp.float32)
         + b_ref[...])
    o_ref[...] = _bn_relu(h, g_ref[...], beta_ref[...])


def _stage2_body(p0_ref, p1_ref, w_ref, b_ref, g_ref, beta_ref, o_ref):
    agg = p0_ref[...] + p1_ref[...]
    h = jnp.dot(agg, w_ref[...], preferred_element_type=jnp.float32) + b_ref[...]
    o_ref[...] = _bn_relu(h, g_ref[...], beta_ref[...])


def _stage3_body(p0_ref, p1_ref, wt_ref, bt_ref, wl_ref, bl_ref, g2_ref,
                 beta2_ref, g3_ref, beta3_ref, o_ref):
    agg = p0_ref[...] + p1_ref[...]
    h = jnp.dot(agg, wt_ref[...], preferred_element_type=jnp.float32) + bt_ref[...]
    h = _bn_relu(h, g2_ref[...], beta2_ref[...])
    h = jnp.maximum(
        jnp.dot(h, wl_ref[...], preferred_element_type=jnp.float32) + bl_ref[...],
        0.0)
    o_ref[...] = _bn_relu(h, g3_ref[...], beta3_ref[...])


_out_nd = jax.ShapeDtypeStruct((N, D), jnp.float32)
_stage1 = pl.pallas_call(_stage1_body, out_shape=_out_nd)
_stage2 = pl.pallas_call(_stage2_body, out_shape=_out_nd)
_stage3 = pl.pallas_call(_stage3_body, out_shape=_out_nd)


def kernel(x, edge_index, W_unite, b_unite, W_graph, b_graph, W_trans,
           b_trans, W_lin, b_lin, g1, beta1, g2, beta2, g3, beta3):
    ppw = EPW - E // NW
    pad_dst = jnp.broadcast_to(
        N + jnp.arange(ppw, dtype=jnp.int32)[None, :] % (NP - N), (NW, ppw))
    pad = jnp.stack([jnp.zeros((NW, ppw), jnp.int32), pad_dst])
    ei4 = jnp.concatenate(
        [edge_index.reshape(2, NW, E // NW), pad], axis=2).reshape(
        2, NW, ITERS, CHUNK)
    zeros = jnp.zeros((RPT, D), jnp.float32)
    row = lambda v: v.reshape(1, D)

    p = _spmm_sc(x, ei4, zeros)[:, :N]
    h = _stage1(x, p[0], p[1], W_unite[:D], W_unite[D:], row(b_unite),
                row(g1), row(beta1))
    p = _spmm_sc(h, ei4, zeros)[:, :N]
    h = _stage2(p[0], p[1], W_graph, row(b_graph), row(g2), row(beta2))
    p = _spmm_sc(h, ei4, zeros)[:, :N]
    h = _stage3(p[0], p[1], W_trans, row(b_trans), W_lin, row(b_lin),
                row(g2), row(beta2), row(g3), row(beta3))
    return h


# speedup vs baseline: 3.0761x; 3.0761x over previous
import functools

import jax
import jax.numpy as jnp
from jax import lax
from jax.experimental import pallas as pl
from jax.experimental.pallas import tpu as pltpu
from jax.experimental.pallas import tpu_sc as plsc

N = 10000
D = 128
E = 320000

NC = 2
NS = 16
NW = NC * NS
CHUNK = 80
HALVES = 2
I2 = 64
ITERS = HALVES * I2
EPW = ITERS * CHUNK
RPT = 632
NP = RPT * NS


def _spmm_sc_body(h_hbm, ei_hbm, zeros_hbm, out_hbm,
                  src_v, dst_v, acc_sh, rows0, rows1, sem0, sem1):
    rows = (rows0, rows1)
    gsem = (sem0, sem1)
    c = lax.axis_index("c")
    s = lax.axis_index("s")
    wid = c * NS + s
    rbase = s * RPT

    def gstart(i, b):
        pltpu.async_copy(h_hbm.at[src_v.at[i]], rows[b], gsem[b])

    def gwait(b):
        pltpu.make_async_copy(h_hbm.at[src_v.at[0]], rows[b], gsem[b]).wait()

    def scat(i, b):
        pltpu.sync_copy(rows[b], acc_sh.at[dst_v.at[i]], add=True)

    pltpu.sync_copy(zeros_hbm, acc_sh.at[pl.ds(rbase, RPT)])
    plsc.subcore_barrier()

    for half in range(HALVES):
        pltpu.sync_copy(ei_hbm.at[0, wid, half], src_v)
        pltpu.sync_copy(ei_hbm.at[1, wid, half], dst_v)
        gstart(0, 0)

        def body(j, carry):
            for b in range(2):
                i = 2 * j + b
                @pl.when(i < I2 - 1)
                def _():
                    gstart(i + 1, 1 - b)

                gwait(b)
                scat(i, b)
            return carry

        lax.fori_loop(0, I2 // 2, body, 0)

    plsc.subcore_barrier()
    pltpu.sync_copy(acc_sh.at[pl.ds(rbase, RPT)],
                    out_hbm.at[c, pl.ds(rbase, RPT)])


_spmm_sc = pl.kernel(
    _spmm_sc_body,
    out_type=jax.ShapeDtypeStruct((NC, NP, D), jnp.float32),
    mesh=plsc.VectorSubcoreMesh(core_axis_name="c", subcore_axis_name="s"),
    scratch_types=[
        pltpu.VMEM((I2, CHUNK), jnp.int32),
        pltpu.VMEM((I2, CHUNK), jnp.int32),
        pltpu.VMEM_SHARED((NP, D), jnp.float32),
        pltpu.VMEM((CHUNK, D), jnp.float32),
        pltpu.VMEM((CHUNK, D), jnp.float32),
        pltpu.SemaphoreType.DMA,
        pltpu.SemaphoreType.DMA,
    ],
)


def _bn_relu(h, g, b):
    mean = jnp.mean(h, axis=0, keepdims=True)
    var = jnp.mean((h - mean) ** 2, axis=0, keepdims=True)
    return jnp.maximum((h - mean) * lax.rsqrt(var + 1e-5) * g + b, 0.0)


def _stage1_body(x_ref, p0_ref, p1_ref, wa_ref, wb_ref, b_ref, g_ref,
                 beta_ref, o_ref):
    agg = p0_ref[...] + p1_ref[...]
    h = (jnp.dot(x_ref[...], wa_ref[...], preferred_element_type=jnp.float32)
         + jnp.dot(agg, wb_ref[...], preferred_element_type=jnp.float32)
         + b_ref[...])
    o_ref[...] = _bn_relu(h, g_ref[...], beta_ref[...])


def _stage2_body(p0_ref, p1_ref, w_ref, b_ref, g_ref, beta_ref, o_ref):
    agg = p0_ref[...] + p1_ref[...]
    h = jnp.dot(agg, w_ref[...], preferred_element_type=jnp.float32) + b_ref[...]
    o_ref[...] = _bn_relu(h, g_ref[...], beta_ref[...])


def _stage3_body(p0_ref, p1_ref, wt_ref, bt_ref, wl_ref, bl_ref, g2_ref,
                 beta2_ref, g3_ref, beta3_ref, o_ref):
    agg = p0_ref[...] + p1_ref[...]
    h = jnp.dot(agg, wt_ref[...], preferred_element_type=jnp.float32) + bt_ref[...]
    h = _bn_relu(h, g2_ref[...], beta2_ref[...])
    h = jnp.maximum(
        jnp.dot(h, wl_ref[...], preferred_element_type=jnp.float32) + bl_ref[...],
        0.0)
    o_ref[...] = _bn_relu(h, g3_ref[...], beta3_ref[...])


_out_nd = jax.ShapeDtypeStruct((N, D), jnp.float32)
_stage1 = pl.pallas_call(_stage1_body, out_shape=_out_nd)
_stage2 = pl.pallas_call(_stage2_body, out_shape=_out_nd)
_stage3 = pl.pallas_call(_stage3_body, out_shape=_out_nd)


def kernel(x, edge_index, W_unite, b_unite, W_graph, b_graph, W_trans,
           b_trans, W_lin, b_lin, g1, beta1, g2, beta2, g3, beta3):
    ppw = EPW - E // NW
    lanes = jnp.arange(ppw, dtype=jnp.int32)[None, :]
    wids = jnp.arange(NW, dtype=jnp.int32)[:, None]
    pad_src = (wids * 313 + lanes * 7) % N
    pad_dst = N + (wids * 13 + lanes) % (NP - N)
    pad = jnp.stack([pad_src, pad_dst])
    ei5 = jnp.concatenate(
        [edge_index.reshape(2, NW, E // NW), pad], axis=2).reshape(
        2, NW, HALVES, I2, CHUNK)
    zeros = jnp.zeros((RPT, D), jnp.float32)
    row = lambda v: v.reshape(1, D)

    p = _spmm_sc(x, ei5, zeros)[:, :N]
    h = _stage1(x, p[0], p[1], W_unite[:D], W_unite[D:], row(b_unite),
                row(g1), row(beta1))
    p = _spmm_sc(h, ei5, zeros)[:, :N]
    h = _stage2(p[0], p[1], W_graph, row(b_graph), row(g2), row(beta2))
    p = _spmm_sc(h, ei5, zeros)[:, :N]
    h = _stage3(p[0], p[1], W_trans, row(b_trans), W_lin, row(b_lin),
                row(g2), row(beta2), row(g3), row(beta3))
    return h


# speedup vs baseline: 3.5285x; 1.1471x over previous
import functools

import jax
import jax.numpy as jnp
from jax import lax
from jax.experimental import pallas as pl
from jax.experimental.pallas import tpu as pltpu
from jax.experimental.pallas import tpu_sc as plsc

N = 10000
D = 128
E = 320000

NC = 2
NS = 16
NW = NC * NS
CHUNK = 80
HALVES = 2
I2 = 63
ITERS = HALVES * I2
EPW = ITERS * CHUNK
NBUF = 3
RPT = 632
NP = RPT * NS


def _spmm_sc_body(h_hbm, ei_hbm, zeros_hbm, out_hbm,
                  src_v, dst_v, acc_sh, *bufs_and_sems):
    rows = bufs_and_sems[0:NBUF]
    gsem = bufs_and_sems[NBUF:2 * NBUF]
    ssem = bufs_and_sems[2 * NBUF:3 * NBUF]
    c = lax.axis_index("c")
    s = lax.axis_index("s")
    wid = c * NS + s
    rbase = s * RPT

    def gstart(i, b):
        pltpu.async_copy(h_hbm.at[src_v.at[i]], rows[b], gsem[b])

    def gwait(b):
        pltpu.make_async_copy(h_hbm.at[src_v.at[0]], rows[b], gsem[b]).wait()

    def sstart(i, b):
        pltpu.async_copy(rows[b], acc_sh.at[dst_v.at[i]], ssem[b], add=True)

    def swait(b):
        pltpu.make_async_copy(rows[b], acc_sh.at[dst_v.at[0]], ssem[b]).wait()

    pltpu.sync_copy(zeros_hbm, acc_sh.at[pl.ds(rbase, RPT)])
    plsc.subcore_barrier()

    for half in range(HALVES):
        pltpu.sync_copy(ei_hbm.at[0, wid, half], src_v)
        pltpu.sync_copy(ei_hbm.at[1, wid, half], dst_v)
        gstart(0, 0)

        def body(j, carry):
            for b in range(NBUF):
                i = j * NBUF + b
                nb = (b + 1) % NBUF
                @pl.when(i >= NBUF - 1)
                def _():
                    swait(nb)

                @pl.when(i < I2 - 1)
                def _():
                    gstart(i + 1, nb)

                gwait(b)
                sstart(i, b)
            return carry

        lax.fori_loop(0, I2 // NBUF, body, 0)
        for k in range(NBUF - 1, 0, -1):
            swait((I2 - k) % NBUF)

    plsc.subcore_barrier()
    pltpu.sync_copy(acc_sh.at[pl.ds(rbase, RPT)],
                    out_hbm.at[c, pl.ds(rbase, RPT)])


_spmm_sc = pl.kernel(
    _spmm_sc_body,
    out_type=jax.ShapeDtypeStruct((NC, NP, D), jnp.float32),
    mesh=plsc.VectorSubcoreMesh(core_axis_name="c", subcore_axis_name="s"),
    scratch_types=[
        pltpu.VMEM((I2, CHUNK), jnp.int32),
        pltpu.VMEM((I2, CHUNK), jnp.int32),
        pltpu.VMEM_SHARED((NP, D), jnp.float32),
    ] + [pltpu.VMEM((CHUNK, D), jnp.float32)] * NBUF
      + [pltpu.SemaphoreType.DMA] * (2 * NBUF),
)


def _bn_relu(h, g, b):
    mean = jnp.mean(h, axis=0, keepdims=True)
    var = jnp.mean((h - mean) ** 2, axis=0, keepdims=True)
    return jnp.maximum((h - mean) * lax.rsqrt(var + 1e-5) * g + b, 0.0)


def _stage1_body(x_ref, p0_ref, p1_ref, wa_ref, wb_ref, b_ref, g_ref,
                 beta_ref, o_ref):
    agg = p0_ref[...] + p1_ref[...]
    h = (jnp.dot(x_ref[...], wa_ref[...], preferred_element_type=jnp.float32)
         + jnp.dot(agg, wb_ref[...], preferred_element_type=jnp.float32)
         + b_ref[...])
    o_ref[...] = _bn_relu(h, g_ref[...], beta_ref[...])


def _stage2_body(p0_ref, p1_ref, w_ref, b_ref, g_ref, beta_ref, o_ref):
    agg = p0_ref[...] + p1_ref[...]
    h = jnp.dot(agg, w_ref[...], preferred_element_type=jnp.float32) + b_ref[...]
    o_ref[...] = _bn_relu(h, g_ref[...], beta_ref[...])


def _stage3_body(p0_ref, p1_ref, wt_ref, bt_ref, wl_ref, bl_ref, g2_ref,
                 beta2_ref, g3_ref, beta3_ref, o_ref):
    agg = p0_ref[...] + p1_ref[...]
    h = jnp.dot(agg, wt_ref[...], preferred_element_type=jnp.float32) + bt_ref[...]
    h = _bn_relu(h, g2_ref[...], beta2_ref[...])
    h = jnp.maximum(
        jnp.dot(h, wl_ref[...], preferred_element_type=jnp.float32) + bl_ref[...],
        0.0)
    o_ref[...] = _bn_relu(h, g3_ref[...], beta3_ref[...])


_out_nd = jax.ShapeDtypeStruct((N, D), jnp.float32)
_stage1 = pl.pallas_call(_stage1_body, out_shape=_out_nd)
_stage2 = pl.pallas_call(_stage2_body, out_shape=_out_nd)
_stage3 = pl.pallas_call(_stage3_body, out_shape=_out_nd)


def kernel(x, edge_index, W_unite, b_unite, W_graph, b_graph, W_trans,
           b_trans, W_lin, b_lin, g1, beta1, g2, beta2, g3, beta3):
    ppw = EPW - E // NW
    lanes = jnp.arange(ppw, dtype=jnp.int32)[None, :]
    wids = jnp.arange(NW, dtype=jnp.int32)[:, None]
    pad_src = (wids * 313 + lanes * 7) % N
    pad_dst = N + (wids * 13 + lanes) % (NP - N)
    pad = jnp.stack([pad_src, pad_dst])
    ei5 = jnp.concatenate(
        [edge_index.reshape(2, NW, E // NW), pad], axis=2).reshape(
        2, NW, HALVES, I2, CHUNK)
    zeros = jnp.zeros((RPT, D), jnp.float32)
    row = lambda v: v.reshape(1, D)

    p = _spmm_sc(x, ei5, zeros)[:, :N]
    h = _stage1(x, p[0], p[1], W_unite[:D], W_unite[D:], row(b_unite),
                row(g1), row(beta1))
    p = _spmm_sc(h, ei5, zeros)[:, :N]
    h = _stage2(p[0], p[1], W_graph, row(b_graph), row(g2), row(beta2))
    p = _spmm_sc(h, ei5, zeros)[:, :N]
    h = _stage3(p[0], p[1], W_trans, row(b_trans), W_lin, row(b_lin),
                row(g2), row(beta2), row(g3), row(beta3))
    return h
